# baseline (speedup 1.0000x reference)
import functools

import jax
import jax.numpy as jnp
from jax import lax
from jax.experimental import pallas as pl
from jax.experimental.pallas import tpu as pltpu
from jax.experimental.pallas import tpu_sc as plsc

EMBED_DIM = 64
BATCH = 4096
HIST_LEN = 200

ROWS = BATCH * HIST_LEN
GROUP = 128
NGROUPS = ROWS // GROUP
NWORKERS = 32
GPW = NGROUPS // NWORKERS
K = 8
NCHUNKS = GPW // K


def _build():
  mesh = plsc.VectorSubcoreMesh(core_axis_name="c", subcore_axis_name="s")

  @functools.partial(
      pl.kernel,
      mesh=mesh,
      out_type=jax.ShapeDtypeStruct((NGROUPS, GROUP, EMBED_DIM), jnp.float32),
      scratch_types=[
          pltpu.VMEM((K, GROUP), jnp.int32),
          pltpu.VMEM((K, GROUP, EMBED_DIM), jnp.float32),
          pltpu.SemaphoreType.DMA,
      ],
      compiler_params=pltpu.CompilerParams(use_tc_tiling_on_sc=False),
  )
  def gather_kernel(idx_hbm, table_hbm, out_hbm, idx_v, rows_v, sem):
    wid = lax.axis_index("s") * 2 + lax.axis_index("c")
    base = wid * GPW

    def chunk(c, carry):
      g0 = base + c * K
      pltpu.sync_copy(idx_hbm.at[pl.ds(g0, K)], idx_v)
      copies = [
          pltpu.async_copy(table_hbm.at[idx_v.at[j]], rows_v.at[j], sem)
          for j in range(K)
      ]
      for cp in copies:
        cp.wait()
      pltpu.sync_copy(rows_v, out_hbm.at[pl.ds(g0, K)])
      return carry

    lax.fori_loop(0, NCHUNKS, chunk, 0)

  return gather_kernel


_gather = _build()


def kernel(language_f, narration_embeds):
  idx = language_f.reshape(NGROUPS, GROUP)
  out = _gather(idx, narration_embeds)
  return out.reshape(BATCH, HIST_LEN, EMBED_DIM)


# speedup vs baseline: 1.0141x; 1.0141x over previous
import functools

import jax
import jax.numpy as jnp
from jax import lax
from jax.experimental import pallas as pl
from jax.experimental.pallas import tpu as pltpu
from jax.experimental.pallas import tpu_sc as plsc

EMBED_DIM = 64
BATCH = 4096
HIST_LEN = 200

ROWS = BATCH * HIST_LEN
GROUP = 128
NGROUPS = ROWS // GROUP
NWORKERS = 32
GPW = NGROUPS // NWORKERS
K = 5
NCHUNKS = GPW // K


def _build():
  mesh = plsc.VectorSubcoreMesh(core_axis_name="c", subcore_axis_name="s")

  @functools.partial(
      pl.kernel,
      mesh=mesh,
      out_type=jax.ShapeDtypeStruct((NGROUPS, GROUP, EMBED_DIM), jnp.float32),
      scratch_types=[
          pltpu.VMEM((GPW, GROUP), jnp.int32),
          pltpu.VMEM((2, K, GROUP, EMBED_DIM), jnp.float32),
          pltpu.SemaphoreType.DMA,
          pltpu.SemaphoreType.DMA,
      ],
      compiler_params=pltpu.CompilerParams(use_tc_tiling_on_sc=False),
  )
  def gather_kernel(idx_hbm, table_hbm, out_hbm, idx_v, rows_v, sem0, sem1):
    wid = lax.axis_index("s") * 2 + lax.axis_index("c")
    base = wid * GPW
    sems = (sem0, sem1)

    pltpu.sync_copy(idx_hbm.at[pl.ds(base, GPW)], idx_v)

    def fire(c, slot):
      for j in range(K):
        pltpu.async_copy(
            table_hbm.at[idx_v.at[c * K + j]], rows_v.at[slot].at[j],
            sems[slot])

    def drain_write(c, slot):
      dst = out_hbm.at[pl.ds(base + c * K, K)]
      pltpu.make_async_copy(dst, rows_v.at[slot], sems[slot]).wait()
      pltpu.sync_copy(rows_v.at[slot], dst)

    fire(0, 0)

    def body(i, carry):
      c0 = 2 * i
      fire(c0 + 1, 1)
      drain_write(c0, 0)

      @pl.when(c0 + 2 < NCHUNKS)
      def _():
        fire(c0 + 2, 0)

      drain_write(c0 + 1, 1)
      return carry

    lax.fori_loop(0, NCHUNKS // 2, body, 0)

  return gather_kernel


_gather = _build()


def kernel(language_f, narration_embeds):
  idx = language_f.reshape(NGROUPS, GROUP)
  out = _gather(idx, narration_embeds)
  return out.reshape(BATCH, HIST_LEN, EMBED_DIM)
